# baseline (speedup 1.0000x reference)
import functools

import jax
import jax.numpy as jnp
from jax.experimental import pallas as pl
from jax.experimental.pallas import tpu as pltpu
from jax.experimental.pallas import tpu_sc as plsc

K = 16
RB = 256
CB = 1024


def _knn_body(x_ref, xt_ref, sq_ref, adj_ref, dis_ref, w_ref,
              dv_ref, dtv_ref, dti_ref, *, n_valid, n_pad):
    i = pl.program_id(0)
    xr = x_ref[...]
    sqr = jnp.sum(xr * xr, axis=1, keepdims=True)
    row_id = i * RB + jax.lax.broadcasted_iota(jnp.int32, (RB, 1), 0)

    num_chunks = n_pad // CB

    row_f = row_id.astype(jnp.float32)
    base_f = jax.lax.broadcasted_iota(
        jnp.int32, (RB, CB), 1).astype(jnp.float32)
    iota2k = jax.lax.broadcasted_iota(
        jnp.int32, (RB, 2 * K), 1).astype(jnp.float32)

    def chunk_step(c, carry):
        tvals, tidx = carry
        xc = xt_ref[:, pl.ds(c * CB, CB)]
        sqc = sq_ref[0, pl.ds(c * CB, CB)].reshape(1, CB)
        dot = jax.lax.dot_general(
            xr, xc, (((1,), (0,)), ((), ())),
            preferred_element_type=jnp.float32)
        d = jnp.maximum(sqr + sqc - 2.0 * dot, 0.0)
        col_f = (c * CB).astype(jnp.float32) + base_f
        d = d + jnp.where(col_f == row_f, 1e9, 0.0)
        d = jnp.where(col_f >= n_valid, 2e9, d)

        t15 = tvals[:, K - 1:K]
        cmax = jnp.max(jnp.sum(jnp.where(d < t15, 1.0, 0.0),
                               axis=1, keepdims=True))
        dv_ref[...] = d
        dtv_ref[...] = jnp.full((RB, K), 3e9, jnp.float32)
        dti_ref[...] = jnp.zeros((RB, K), jnp.int32)
        for j in range(K):
            @pl.when(jnp.float32(j) < cmax)
            def _():
                vals = dv_ref[...]
                m = jnp.min(vals, axis=1, keepdims=True)
                posf = jnp.min(jnp.where(vals == m, base_f, 3e9),
                               axis=1, keepdims=True)
                dv_ref[...] = jnp.where(base_f == posf, 3e9, vals)
                dtv_ref[:, pl.ds(j, 1)] = m
                dti_ref[:, pl.ds(j, 1)] = c * CB + posf.astype(jnp.int32)

        mv = jnp.concatenate([tvals, dtv_ref[...]], axis=1)
        mi = jnp.concatenate([tidx, dti_ref[...]], axis=1)
        new_v, new_i = [], []
        for _ in range(K):
            m = jnp.min(mv, axis=1, keepdims=True)
            posf = jnp.min(jnp.where(mv == m, iota2k, 3e9),
                           axis=1, keepdims=True)
            sel = iota2k == posf
            gi = jnp.max(jnp.where(sel, mi, -(1 << 30)),
                         axis=1, keepdims=True)
            mv = jnp.where(sel, 3e9, mv)
            new_v.append(m)
            new_i.append(gi)
        return (jnp.concatenate(new_v, axis=1),
                jnp.concatenate(new_i, axis=1))

    tvals0 = jnp.full((RB, K), 3e9, dtype=jnp.float32)
    tidx0 = jnp.full((RB, K), -1, dtype=jnp.int32)
    tvals, tidx = jax.lax.fori_loop(0, num_chunks, chunk_step,
                                    (tvals0, tidx0))

    dis = jnp.sqrt(jnp.maximum(tvals, 1e-12))
    mx = jnp.max(-dis, axis=1, keepdims=True)
    e = jnp.exp(-dis - mx)
    w = e / jnp.sum(e, axis=1, keepdims=True)

    adj_ref[...] = tidx
    dis_ref[...] = dis
    w_ref[...] = w


def _knn_topk(x_pad, sq_pad, n_valid):
    n_pad, c = x_pad.shape
    xt = x_pad.T
    grid = (n_pad // RB,)
    out_shapes = (
        jax.ShapeDtypeStruct((n_pad, K), jnp.int32),
        jax.ShapeDtypeStruct((n_pad, K), jnp.float32),
        jax.ShapeDtypeStruct((n_pad, K), jnp.float32),
    )
    kfun = functools.partial(_knn_body, n_valid=n_valid, n_pad=n_pad)
    return pl.pallas_call(
        kfun,
        grid=grid,
        in_specs=[
            pl.BlockSpec((RB, c), lambda i: (i, 0)),
            pl.BlockSpec((c, n_pad), lambda i: (0, 0)),
            pl.BlockSpec((1, n_pad), lambda i: (0, 0)),
        ],
        out_specs=(
            pl.BlockSpec((RB, K), lambda i: (i, 0)),
            pl.BlockSpec((RB, K), lambda i: (i, 0)),
            pl.BlockSpec((RB, K), lambda i: (i, 0)),
        ),
        out_shape=out_shapes,
        scratch_shapes=[
            pltpu.VMEM((RB, CB), jnp.float32),
            pltpu.VMEM((RB, K), jnp.float32),
            pltpu.VMEM((RB, K), jnp.int32),
        ],
    )(x_pad, xt, sq_pad.reshape(1, n_pad))


SUB = 4
GROWS = SUB * K
LANES = 16
FV = 128 // LANES


def _agg_sc_body(x_hbm, adjf_hbm, wf_hbm, agg_hbm,
                 adj_v, w_v, buf0, buf1, out_v, sem0, sem1):
    nw = 32
    wid = jax.lax.axis_index("s") * 2 + jax.lax.axis_index("c")
    n_per_w = adj_v.shape[0] // K
    nsub = n_per_w // SUB
    base = wid * n_per_w
    pltpu.sync_copy(adjf_hbm.at[pl.ds(base * K, n_per_w * K)], adj_v)
    pltpu.sync_copy(wf_hbm.at[pl.ds(base * K, n_per_w * K)], w_v)

    def gather(s, buf, sem):
        return pltpu.make_async_copy(
            x_hbm.at[adj_v.at[pl.ds(s * GROWS, GROWS)]], buf, sem)

    def compute(s, buf):
        for n in range(SUB):
            wrow = w_v[pl.ds(s * GROWS + n * K, K)]
            for j in range(FV):
                acc = jnp.zeros((LANES,), jnp.float32)
                for k in range(K):
                    acc = acc + wrow[k] * buf[n * K + k, pl.ds(j * LANES, LANES)]
                out_v[n, pl.ds(j * LANES, LANES)] = acc
        pltpu.sync_copy(out_v, agg_hbm.at[pl.ds(base + s * SUB, SUB)])

    gather(0, buf0, sem0).start()

    def step(t, _):
        s0 = 2 * t
        s1 = 2 * t + 1
        gather(s1, buf1, sem1).start()
        gather(s0, buf0, sem0).wait()
        compute(s0, buf0)

        @pl.when(t < nsub // 2 - 1)
        def _():
            gather(s0 + 2, buf0, sem0).start()

        gather(s1, buf1, sem1).wait()
        compute(s1, buf1)
        return 0

    jax.lax.fori_loop(0, nsub // 2, step, 0)


def _agg_sparsecore(x, adj_p, w_p):
    n_pad = adj_p.shape[0]
    c = x.shape[1]
    n_per_w = n_pad // 32
    mesh = plsc.VectorSubcoreMesh(core_axis_name="c", subcore_axis_name="s")
    kfun = pl.kernel(
        _agg_sc_body,
        out_type=jax.ShapeDtypeStruct((n_pad, c), jnp.float32),
        mesh=mesh,
        scratch_types=[
            pltpu.VMEM((n_per_w * K,), jnp.int32),
            pltpu.VMEM((n_per_w * K,), jnp.float32),
            pltpu.VMEM((GROWS, c), jnp.float32),
            pltpu.VMEM((GROWS, c), jnp.float32),
            pltpu.VMEM((SUB, c), jnp.float32),
            pltpu.SemaphoreType.DMA,
            pltpu.SemaphoreType.DMA,
        ],
    )
    return kfun(x, adj_p.reshape(-1), w_p.reshape(-1))


def _linear_body(agg_ref, w_ref, b_ref, out_ref):
    acc = jax.lax.dot_general(
        agg_ref[...], w_ref[...], (((1,), (0,)), ((), ())),
        preferred_element_type=jnp.float32)
    out_ref[...] = jnp.maximum(acc + b_ref[...], 0.0)


def _linear_relu(agg, w, b):
    n, c = agg.shape
    out = w.shape[1]
    rb = 2000 if n % 2000 == 0 else n
    return pl.pallas_call(
        _linear_body,
        grid=(n // rb,),
        in_specs=[
            pl.BlockSpec((rb, c), lambda i: (i, 0)),
            pl.BlockSpec((c, out), lambda i: (0, 0)),
            pl.BlockSpec((1, out), lambda i: (0, 0)),
        ],
        out_specs=pl.BlockSpec((rb, out), lambda i: (i, 0)),
        out_shape=jax.ShapeDtypeStruct((n, out), jnp.float32),
    )(agg, w, b.reshape(1, out))


def kernel(X, W, b):
    n, c = X.shape
    n_pad = ((n + RB - 1) // RB) * RB
    if n_pad % CB:
        n_pad = ((n_pad + CB - 1) // CB) * CB
    x_pad = jnp.pad(X, ((0, n_pad - n), (0, 0)))
    sq_pad = jnp.pad(jnp.sum(X * X, axis=1), (0, n_pad - n))

    adj_p, dis_p, w_p = _knn_topk(x_pad, sq_pad, n)
    adj, dis = adj_p[:n], dis_p[:n]

    agg = _agg_sparsecore(X, adj_p, w_p)[:n]

    out = _linear_relu(agg, W, b)
    return (out, adj, dis)


# speedup vs baseline: 1.5815x; 1.5815x over previous
import functools

import jax
import jax.numpy as jnp
from jax.experimental import pallas as pl
from jax.experimental.pallas import tpu as pltpu
from jax.experimental.pallas import tpu_sc as plsc

K = 16
RB = 256
CB = 5120


def _knn_body(x_ref, xt_ref, sq_ref, adj_ref, dis_ref, w_ref,
              *, n_valid, n_pad):
    i = pl.program_id(0)
    xr = x_ref[...]
    sqr = jnp.sum(xr * xr, axis=1, keepdims=True)
    row_id = i * RB + jax.lax.broadcasted_iota(jnp.int32, (RB, 1), 0)

    num_chunks = n_pad // CB

    row_f = row_id.astype(jnp.float32)
    base_f = jax.lax.broadcasted_iota(
        jnp.int32, (RB, CB), 1).astype(jnp.float32)
    kiota = jax.lax.broadcasted_iota(jnp.int32, (RB, K), 1)
    pos_iota = jax.lax.broadcasted_iota(
        jnp.int32, (RB, K + CB), 1).astype(jnp.float32)

    def chunk_step(c, carry):
        tvals, tidx = carry
        xc = xt_ref[:, pl.ds(c * CB, CB)]
        sqc = sq_ref[0, pl.ds(c * CB, CB)].reshape(1, CB)
        dot = jax.lax.dot_general(
            xr, xc, (((1,), (0,)), ((), ())),
            preferred_element_type=jnp.float32)
        d = jnp.maximum(sqr + sqc - 2.0 * dot, 0.0)
        col_f = (c * CB).astype(jnp.float32) + base_f
        d = d + jnp.where(col_f == row_f, 1e9, 0.0)
        d = jnp.where(col_f >= n_valid, 2e9, d)

        vals = jnp.concatenate([tvals, d], axis=1)
        new_v, new_i = [], []
        for _ in range(K):
            m = jnp.min(vals, axis=1, keepdims=True)
            posf = jnp.min(jnp.where(vals == m, pos_iota, 3e9),
                           axis=1, keepdims=True)
            vals = jnp.where(pos_iota == posf, 3e9, vals)
            pos_i = posf.astype(jnp.int32)
            g_t = jnp.max(jnp.where(kiota == pos_i, tidx, -(1 << 30)),
                          axis=1, keepdims=True)
            gi = jnp.where(pos_i < K, g_t, c * CB + pos_i - K)
            new_v.append(m)
            new_i.append(gi)
        return (jnp.concatenate(new_v, axis=1),
                jnp.concatenate(new_i, axis=1))

    tvals0 = jnp.full((RB, K), 3e9, dtype=jnp.float32)
    tidx0 = jnp.full((RB, K), -1, dtype=jnp.int32)
    tvals, tidx = jax.lax.fori_loop(0, num_chunks, chunk_step,
                                    (tvals0, tidx0))

    dis = jnp.sqrt(jnp.maximum(tvals, 1e-12))
    mx = jnp.max(-dis, axis=1, keepdims=True)
    e = jnp.exp(-dis - mx)
    w = e / jnp.sum(e, axis=1, keepdims=True)

    adj_ref[...] = tidx
    dis_ref[...] = dis
    w_ref[...] = w


def _knn_topk(x_pad, sq_pad, n_valid):
    n_pad, c = x_pad.shape
    xt = x_pad.T
    grid = (n_pad // RB,)
    out_shapes = (
        jax.ShapeDtypeStruct((n_pad, K), jnp.int32),
        jax.ShapeDtypeStruct((n_pad, K), jnp.float32),
        jax.ShapeDtypeStruct((n_pad, K), jnp.float32),
    )
    kfun = functools.partial(_knn_body, n_valid=n_valid, n_pad=n_pad)
    return pl.pallas_call(
        kfun,
        grid=grid,
        in_specs=[
            pl.BlockSpec((RB, c), lambda i: (i, 0)),
            pl.BlockSpec((c, n_pad), lambda i: (0, 0)),
            pl.BlockSpec((1, n_pad), lambda i: (0, 0)),
        ],
        out_specs=(
            pl.BlockSpec((RB, K), lambda i: (i, 0)),
            pl.BlockSpec((RB, K), lambda i: (i, 0)),
            pl.BlockSpec((RB, K), lambda i: (i, 0)),
        ),
        out_shape=out_shapes,
    )(x_pad, xt, sq_pad.reshape(1, n_pad))


SUB = 4
GROWS = SUB * K
LANES = 16
FV = 128 // LANES


def _agg_sc_body(x_hbm, adjf_hbm, wf_hbm, agg_hbm,
                 adj_v, w_v, buf0, buf1, out_v, sem0, sem1):
    nw = 32
    wid = jax.lax.axis_index("s") * 2 + jax.lax.axis_index("c")
    n_per_w = adj_v.shape[0] // K
    nsub = n_per_w // SUB
    base = wid * n_per_w
    pltpu.sync_copy(adjf_hbm.at[pl.ds(base * K, n_per_w * K)], adj_v)
    pltpu.sync_copy(wf_hbm.at[pl.ds(base * K, n_per_w * K)], w_v)

    def gather(s, buf, sem):
        return pltpu.make_async_copy(
            x_hbm.at[adj_v.at[pl.ds(s * GROWS, GROWS)]], buf, sem)

    def compute(s, buf):
        for n in range(SUB):
            wrow = w_v[pl.ds(s * GROWS + n * K, K)]
            for j in range(FV):
                acc = jnp.zeros((LANES,), jnp.float32)
                for k in range(K):
                    acc = acc + wrow[k] * buf[n * K + k, pl.ds(j * LANES, LANES)]
                out_v[n, pl.ds(j * LANES, LANES)] = acc
        pltpu.sync_copy(out_v, agg_hbm.at[pl.ds(base + s * SUB, SUB)])

    gather(0, buf0, sem0).start()

    def step(t, _):
        s0 = 2 * t
        s1 = 2 * t + 1
        gather(s1, buf1, sem1).start()
        gather(s0, buf0, sem0).wait()
        compute(s0, buf0)

        @pl.when(t < nsub // 2 - 1)
        def _():
            gather(s0 + 2, buf0, sem0).start()

        gather(s1, buf1, sem1).wait()
        compute(s1, buf1)
        return 0

    jax.lax.fori_loop(0, nsub // 2, step, 0)


def _agg_sparsecore(x, adj_p, w_p):
    n_pad = adj_p.shape[0]
    c = x.shape[1]
    n_per_w = n_pad // 32
    mesh = plsc.VectorSubcoreMesh(core_axis_name="c", subcore_axis_name="s")
    kfun = pl.kernel(
        _agg_sc_body,
        out_type=jax.ShapeDtypeStruct((n_pad, c), jnp.float32),
        mesh=mesh,
        scratch_types=[
            pltpu.VMEM((n_per_w * K,), jnp.int32),
            pltpu.VMEM((n_per_w * K,), jnp.float32),
            pltpu.VMEM((GROWS, c), jnp.float32),
            pltpu.VMEM((GROWS, c), jnp.float32),
            pltpu.VMEM((SUB, c), jnp.float32),
            pltpu.SemaphoreType.DMA,
            pltpu.SemaphoreType.DMA,
        ],
    )
    return kfun(x, adj_p.reshape(-1), w_p.reshape(-1))


def _linear_body(agg_ref, w_ref, b_ref, out_ref):
    acc = jax.lax.dot_general(
        agg_ref[...], w_ref[...], (((1,), (0,)), ((), ())),
        preferred_element_type=jnp.float32)
    out_ref[...] = jnp.maximum(acc + b_ref[...], 0.0)


def _linear_relu(agg, w, b):
    n, c = agg.shape
    out = w.shape[1]
    rb = 2000 if n % 2000 == 0 else n
    return pl.pallas_call(
        _linear_body,
        grid=(n // rb,),
        in_specs=[
            pl.BlockSpec((rb, c), lambda i: (i, 0)),
            pl.BlockSpec((c, out), lambda i: (0, 0)),
            pl.BlockSpec((1, out), lambda i: (0, 0)),
        ],
        out_specs=pl.BlockSpec((rb, out), lambda i: (i, 0)),
        out_shape=jax.ShapeDtypeStruct((n, out), jnp.float32),
    )(agg, w, b.reshape(1, out))


def kernel(X, W, b):
    n, c = X.shape
    n_pad = ((n + RB - 1) // RB) * RB
    if n_pad % CB:
        n_pad = ((n_pad + CB - 1) // CB) * CB
    x_pad = jnp.pad(X, ((0, n_pad - n), (0, 0)))
    sq_pad = jnp.pad(jnp.sum(X * X, axis=1), (0, n_pad - n))

    adj_p, dis_p, w_p = _knn_topk(x_pad, sq_pad, n)
    adj, dis = adj_p[:n], dis_p[:n]

    agg = _agg_sparsecore(X, adj_p, w_p)[:n]

    out = _linear_relu(agg, W, b)
    return (out, adj, dis)
